# baseline (speedup 1.0000x reference)
import functools

import jax
import jax.numpy as jnp
import numpy as np
from jax import lax
from jax.experimental import pallas as pl
from jax.experimental.pallas import tpu as pltpu
from jax.experimental.pallas import tpu_sc as plsc

_EPS = 1e-5
_NUM_SUBCORES = 32


def _fps_body(b, n, npoint, x_ref, y_ref, z_ref, ox_ref, oy_ref, oz_ref):
    n8 = n // 8
    s8 = max(npoint // 8, 1)
    x = x_ref[...]
    y = y_ref[...]
    z = z_ref[...]
    iota_n = (lax.broadcasted_iota(jnp.int32, (b, 8, n8), 1) * n8
              + lax.broadcasted_iota(jnp.int32, (b, 8, n8), 2))
    iota_s = (lax.broadcasted_iota(jnp.int32, (b, 8, s8), 1) * s8
              + lax.broadcasted_iota(jnp.int32, (b, 8, s8), 2))

    def body(i, state):
        dist, far, ox, oy, oz = state
        sel = iota_n == far
        cx = jnp.sum(jnp.where(sel, x, 0.0), axis=(1, 2), keepdims=True)
        cy = jnp.sum(jnp.where(sel, y, 0.0), axis=(1, 2), keepdims=True)
        cz = jnp.sum(jnp.where(sel, z, 0.0), axis=(1, 2), keepdims=True)
        ox = jnp.where(iota_s == i, cx, ox)
        oy = jnp.where(iota_s == i, cy, oy)
        oz = jnp.where(iota_s == i, cz, oz)
        dx = x - cx
        dy = y - cy
        dz = z - cz
        d = (dx * dx + dy * dy) + dz * dz
        dist = jnp.minimum(dist, d)
        m = jnp.max(dist, axis=(1, 2), keepdims=True)
        far = jnp.min(jnp.where(dist == m, iota_n, n), axis=(1, 2),
                      keepdims=True)
        return dist, far, ox, oy, oz

    init = (jnp.full((b, 8, n8), 1e10, jnp.float32),
            jnp.zeros((b, 1, 1), jnp.int32),
            jnp.zeros((b, 8, s8), jnp.float32),
            jnp.zeros((b, 8, s8), jnp.float32),
            jnp.zeros((b, 8, s8), jnp.float32))
    _, _, ox, oy, oz = lax.fori_loop(0, npoint, body, init)
    ox_ref[...] = ox
    oy_ref[...] = oy
    oz_ref[...] = oz


def fps_pallas(xyz, npoint):
    b, n, _ = xyz.shape
    n8 = n // 8
    s8 = max(npoint // 8, 1)
    xr = xyz[..., 0].reshape(b, 8, n8)
    yr = xyz[..., 1].reshape(b, 8, n8)
    zr = xyz[..., 2].reshape(b, 8, n8)
    out_sh = jax.ShapeDtypeStruct((b, 8, s8), jnp.float32)
    ox, oy, oz = pl.pallas_call(
        functools.partial(_fps_body, b, n, npoint),
        out_shape=[out_sh, out_sh, out_sh],
    )(xr, yr, zr)
    return jnp.stack([ox.reshape(b, npoint), oy.reshape(b, npoint),
                      oz.reshape(b, npoint)], axis=-1)


def _sqdist_body(src_ref, dstt_ref, out_ref):
    src = src_ref[0]
    dstt = dstt_ref[0]
    sb0 = src[:, 0:1]
    sb1 = src[:, 1:2]
    sb2 = src[:, 2:3]
    ss = (sb0 * sb0 + sb1 * sb1) + sb2 * sb2
    d0 = dstt[0:1, :]
    d1 = dstt[1:2, :]
    d2 = dstt[2:3, :]
    sd = (d0 * d0 + d1 * d1) + d2 * d2
    dot = jnp.dot(src.astype(jnp.bfloat16), dstt.astype(jnp.bfloat16),
                  preferred_element_type=jnp.float32)
    out_ref[0] = (ss + sd) - 2.0 * dot


def sqdist_pallas(new_xyz, xyz, s_blk=None):
    b, s, _ = new_xyz.shape
    n = xyz.shape[1]
    if s_blk is None:
        s_blk = min(s, 256)
    xyz_t = jnp.transpose(xyz, (0, 2, 1))
    return pl.pallas_call(
        _sqdist_body,
        grid=(b, s // s_blk),
        in_specs=[pl.BlockSpec((1, s_blk, 3), lambda i, j: (i, j, 0)),
                  pl.BlockSpec((1, 3, n), lambda i, j: (i, 0, 0))],
        out_specs=pl.BlockSpec((1, s_blk, n), lambda i, j: (i, j, 0)),
        out_shape=jax.ShapeDtypeStruct((b, s, n), jnp.float32),
    )(new_xyz, xyz_t)


def _select_indices(drow_ref, idx_refs, r2s, ks, n, joff, iota16):
    nb_total = n // 16
    nbr = len(r2s)

    fill = jnp.full((16,), n - 1, jnp.int32) + joff
    for r in range(nbr):
        idx_refs[r][pl.ds(0, 16)] = fill

    def scan_blk(nb, cnts):
        dv = drow_ref[pl.ds(nb * 16, 16)]
        jv = joff + nb * 16 + iota16
        new = []
        for r in range(nbr):
            mask = dv <= r2s[r]
            cnt = cnts[r]

            @pl.when(cnt < ks[r])
            def _(idx_ref=idx_refs[r], cnt=cnt, jv=jv, mask=mask):
                plsc.store_compressed(idx_ref.at[pl.ds(cnt, 16)], jv,
                                      mask=mask)

            new.append(cnt + jnp.sum(mask.astype(jnp.int32)))
        return tuple(new)

    cnts = lax.fori_loop(0, nb_total, scan_blk,
                         tuple(jnp.int32(0) for _ in range(nbr)))

    for r in range(nbr):
        v0 = idx_refs[r][pl.ds(0, 16)]
        s0 = jnp.min(jnp.where(iota16 == 0, v0, jnp.int32(2147483647)))
        first = s0 + jnp.zeros((16,), jnp.int32)
        for p in range(ks[r] // 16):
            iv = cnts[r] + p * 16 + iota16
            plsc.store_scatter(idx_refs[r], [iv], first, mask=iv < ks[r])
    return cnts


def _sc_sa1_body(n, s, cpw, r2s, ks,
                 dd, xh, yh, zh, cxh, cyh, czh,
                 out0, out1, out2,
                 drow, xloc, yloc, zloc, ccx, ccy, ccz,
                 idx0, idx1, idx2, rowbuf):
    outs = [out0, out1, out2]
    idx_refs = [idx0, idx1, idx2]
    w = lax.axis_index("s") * 2 + lax.axis_index("c")
    base_c = w * cpw
    b = base_c // s
    pltpu.sync_copy(xh.at[b], xloc)
    pltpu.sync_copy(yh.at[b], yloc)
    pltpu.sync_copy(zh.at[b], zloc)
    pltpu.sync_copy(cxh.at[pl.ds(base_c, cpw)], ccx)
    pltpu.sync_copy(cyh.at[pl.ds(base_c, cpw)], ccy)
    pltpu.sync_copy(czh.at[pl.ds(base_c, cpw)], ccz)
    iota16 = lax.iota(jnp.int32, 16)

    def per_center(cl, carry):
        c = base_c + cl
        pltpu.sync_copy(dd.at[c], drow)
        clv = jnp.full((16,), 0, jnp.int32) + cl
        cxs = plsc.load_gather(ccx, [clv])
        cys = plsc.load_gather(ccy, [clv])
        czs = plsc.load_gather(ccz, [clv])
        _select_indices(drow, idx_refs, r2s, ks, n, jnp.int32(0), iota16)
        for r in range(3):
            k = ks[r]
            for p in range(k // 16):
                kv = idx_refs[r][pl.ds(p * 16, 16)]
                gx = plsc.load_gather(xloc, [kv])
                gy = plsc.load_gather(yloc, [kv])
                gz = plsc.load_gather(zloc, [kv])
                basei = (p * 16 + iota16) * 6
                plsc.store_scatter(rowbuf, [basei], gx)
                plsc.store_scatter(rowbuf, [basei + 1], gy)
                plsc.store_scatter(rowbuf, [basei + 2], gz)
                plsc.store_scatter(rowbuf, [basei + 3], gx - cxs)
                plsc.store_scatter(rowbuf, [basei + 4], gy - cys)
                plsc.store_scatter(rowbuf, [basei + 5], gz - czs)
            pltpu.sync_copy(rowbuf.at[pl.ds(0, k * 6)],
                            outs[r].at[pl.ds(c * (k * 6), k * 6)])
        return carry

    lax.fori_loop(0, cpw, per_center, jnp.int32(0))


def sc_group_sa1(d, xyz, new_xyz, r2s, ks):
    bs, n = d.shape
    b3 = xyz.shape[0]
    s = bs // b3
    cpw = bs // _NUM_SUBCORES
    kmax = max(ks)
    mesh = plsc.VectorSubcoreMesh(core_axis_name="c", subcore_axis_name="s")
    out_type = [jax.ShapeDtypeStruct((bs * k * 6,), jnp.float32) for k in ks]
    scratch = [
        pltpu.VMEM((n,), jnp.float32),
        pltpu.VMEM((n,), jnp.float32),
        pltpu.VMEM((n,), jnp.float32),
        pltpu.VMEM((n,), jnp.float32),
        pltpu.VMEM((cpw,), jnp.float32),
        pltpu.VMEM((cpw,), jnp.float32),
        pltpu.VMEM((cpw,), jnp.float32),
        pltpu.VMEM((ks[0] + 16,), jnp.int32),
        pltpu.VMEM((ks[1] + 16,), jnp.int32),
        pltpu.VMEM((ks[2] + 16,), jnp.int32),
        pltpu.VMEM((kmax * 6,), jnp.float32),
    ]
    fn = pl.kernel(
        functools.partial(_sc_sa1_body, n, s, cpw, r2s, ks),
        out_type=out_type, mesh=mesh, scratch_types=scratch,
        compiler_params=pltpu.CompilerParams(needs_layout_passes=False))
    return fn(d,
              xyz[..., 0], xyz[..., 1], xyz[..., 2],
              new_xyz[:, 0], new_xyz[:, 1], new_xyz[:, 2])


def _sc_sa2_body(n, s, cpw, c1, r2s, ks,
                 dd, t0, t1, u0, u1,
                 y0, y1,
                 drow, idx0, idx1, idxk0, idxk1, urow, rows0, rows1, sem):
    ts = [t0, t1]
    us = [u0, u1]
    ys = [y0, y1]
    idx_refs = [idx0, idx1]
    idxk_refs = [idxk0, idxk1]
    rows_refs = [rows0, rows1]
    w = lax.axis_index("s") * 2 + lax.axis_index("c")
    base_c = w * cpw
    iota16 = lax.iota(jnp.int32, 16)
    nch = c1 // 16

    def per_center(cl, carry):
        c = base_c + cl
        b = c // s
        joff = b * n
        pltpu.sync_copy(dd.at[c], drow)
        _select_indices(drow, idx_refs, r2s, ks, n, joff, iota16)
        for r in range(2):
            k = ks[r]
            for p in range(k // 16):
                idxk_refs[r][pl.ds(p * 16, 16)] = \
                    idx_refs[r][pl.ds(p * 16, 16)]
            pltpu.async_copy(ts[r].at[idxk_refs[r]], rows_refs[r], sem).wait()
            pltpu.sync_copy(us[r].at[c], urow)

            def sub_row(kk, carry2, rows_ref=rows_refs[r]):
                rr = rows_ref.at[kk]
                for ch in range(nch):
                    sl = pl.ds(ch * 16, 16)
                    rr[sl] = rr[sl] - urow[sl]
                return carry2

            lax.fori_loop(0, k, sub_row, jnp.int32(0))
            pltpu.sync_copy(rows_refs[r], ys[r].at[pl.ds(c * k, k)])
        return carry

    lax.fori_loop(0, cpw, per_center, jnp.int32(0))


def sc_group_sa2(d, t_tabs, u_tabs, r2s, ks):
    bs, n = d.shape
    c1 = t_tabs[0].shape[1]
    bt = t_tabs[0].shape[0] // n
    s = bs // bt
    cpw = bs // _NUM_SUBCORES
    mesh = plsc.VectorSubcoreMesh(core_axis_name="c", subcore_axis_name="s")
    out_type = [jax.ShapeDtypeStruct((bs * k, c1), jnp.float32) for k in ks]
    scratch = [
        pltpu.VMEM((n,), jnp.float32),
        pltpu.VMEM((ks[0] + 16,), jnp.int32),
        pltpu.VMEM((ks[1] + 16,), jnp.int32),
        pltpu.VMEM((ks[0],), jnp.int32),
        pltpu.VMEM((ks[1],), jnp.int32),
        pltpu.VMEM((c1,), jnp.float32),
        pltpu.VMEM((ks[0], c1), jnp.float32),
        pltpu.VMEM((ks[1], c1), jnp.float32),
        pltpu.SemaphoreType.DMA,
    ]
    fn = pl.kernel(
        functools.partial(_sc_sa2_body, n, s, cpw, c1, r2s, ks),
        out_type=out_type, mesh=mesh, scratch_types=scratch,
        compiler_params=pltpu.CompilerParams(needs_layout_passes=False))
    return fn(d, t_tabs[0], t_tabs[1], u_tabs[0], u_tabs[1])


def _layer_body(apply_relu, x_ref, a_ref, d_ref, wt_ref, b_ref,
                y_ref, s1_ref, s2_ref):
    x = x_ref[...]
    x = x * a_ref[...] + d_ref[...]
    if apply_relu:
        x = jnp.maximum(x, 0.0)
    y = jnp.dot(x.astype(jnp.bfloat16), wt_ref[...].astype(jnp.bfloat16),
                preferred_element_type=jnp.float32) + b_ref[...]
    y_ref[...] = y
    s1 = jnp.sum(y, axis=0, keepdims=True)
    s2 = jnp.sum(y * y, axis=0, keepdims=True)

    @pl.when(pl.program_id(0) == 0)
    def _():
        s1_ref[...] = s1
        s2_ref[...] = s2

    @pl.when(pl.program_id(0) != 0)
    def _():
        s1_ref[...] = s1_ref[...] + s1
        s2_ref[...] = s2_ref[...] + s2


def layer_pallas(x, a, d, wt, bias, apply_relu, blk=2048):
    r, cin = x.shape
    cout = wt.shape[1]
    blk = min(blk, r)
    assert r % blk == 0
    grid = r // blk
    return pl.pallas_call(
        functools.partial(_layer_body, apply_relu),
        grid=(grid,),
        in_specs=[pl.BlockSpec((blk, cin), lambda i: (i, 0)),
                  pl.BlockSpec((1, cin), lambda i: (0, 0)),
                  pl.BlockSpec((1, cin), lambda i: (0, 0)),
                  pl.BlockSpec((cin, cout), lambda i: (0, 0)),
                  pl.BlockSpec((1, cout), lambda i: (0, 0))],
        out_specs=[pl.BlockSpec((blk, cout), lambda i: (i, 0)),
                   pl.BlockSpec((1, cout), lambda i: (0, 0)),
                   pl.BlockSpec((1, cout), lambda i: (0, 0))],
        out_shape=[jax.ShapeDtypeStruct((r, cout), jnp.float32),
                   jax.ShapeDtypeStruct((1, cout), jnp.float32),
                   jax.ShapeDtypeStruct((1, cout), jnp.float32)],
    )(x, a, d, wt, bias)


def _stats_body(y_ref, s1_ref, s2_ref):
    y = y_ref[...]
    s1 = jnp.sum(y, axis=0, keepdims=True)
    s2 = jnp.sum(y * y, axis=0, keepdims=True)

    @pl.when(pl.program_id(0) == 0)
    def _():
        s1_ref[...] = s1
        s2_ref[...] = s2

    @pl.when(pl.program_id(0) != 0)
    def _():
        s1_ref[...] = s1_ref[...] + s1
        s2_ref[...] = s2_ref[...] + s2


def stats_pallas(y, blk=2048):
    r, c = y.shape
    blk = min(blk, r)
    grid = r // blk
    return pl.pallas_call(
        _stats_body,
        grid=(grid,),
        in_specs=[pl.BlockSpec((blk, c), lambda i: (i, 0))],
        out_specs=[pl.BlockSpec((1, c), lambda i: (0, 0)),
                   pl.BlockSpec((1, c), lambda i: (0, 0))],
        out_shape=[jax.ShapeDtypeStruct((1, c), jnp.float32),
                   jax.ShapeDtypeStruct((1, c), jnp.float32)],
    )(y)


def _maxpool_body(m, k, y_ref, a_ref, d_ref, out_ref):
    x = y_ref[...] * a_ref[...] + d_ref[...]
    x = jnp.maximum(x, 0.0)
    x = x.reshape(m, k, x.shape[-1])
    out_ref[...] = jnp.max(x, axis=1)


def maxpool_pallas(y, a, d, k, blk_rows=2048):
    r, c = y.shape
    blk_rows = min(blk_rows, r)
    m = blk_rows // k
    grid = r // blk_rows
    return pl.pallas_call(
        functools.partial(_maxpool_body, m, k),
        grid=(grid,),
        in_specs=[pl.BlockSpec((blk_rows, c), lambda i: (i, 0)),
                  pl.BlockSpec((1, c), lambda i: (0, 0)),
                  pl.BlockSpec((1, c), lambda i: (0, 0))],
        out_specs=pl.BlockSpec((m, c), lambda i: (i, 0)),
        out_shape=jax.ShapeDtypeStruct((r // k, c), jnp.float32),
    )(y, a, d)


def _fold_bn(s1, s2, r, gamma, beta):
    mean = s1[0] / r
    var = jnp.maximum(s2[0] / r - mean * mean, 0.0)
    a = gamma / jnp.sqrt(var + _EPS)
    d = beta - mean * a
    return a[None, :], d[None, :]


def _mlp_tail(y1, s1a, s1b, layers, k):
    r = y1.shape[0]
    y, sa, sb = y1, s1a, s1b
    for i in range(1, len(layers)):
        a, d = _fold_bn(sa, sb, r, layers[i - 1][2], layers[i - 1][3])
        y, sa, sb = layer_pallas(y, a, d, layers[i][0], layers[i][1][None, :],
                                 True)
    a, d = _fold_bn(sa, sb, r, layers[-1][2], layers[-1][3])
    return maxpool_pallas(y, a, d, k)


def _prep_layers(layers):
    return [(w.T, b, g, bt) for (w, b, g, bt) in layers]


def _r2(radius):
    return np.float32(np.float64(radius) ** 2)


def kernel(xyz, params):
    b, n, _ = xyz.shape
    s1 = 512
    s2 = 128

    c1 = fps_pallas(xyz, s1)
    d1 = sqdist_pallas(c1, xyz).reshape(b * s1, n)
    r2s1 = (_r2(0.1), _r2(0.2), _r2(0.4))
    ks1 = (32, 64, 128)
    groups = sc_group_sa1(d1, xyz, c1.reshape(b * s1, 3), r2s1, ks1)

    c2 = fps_pallas(c1, s2)
    d2 = sqdist_pallas(c2, c1).reshape(b * s2, s1)
    c2_flat = c2.reshape(b * s2, 3)
    lys2 = [_prep_layers(layers) for layers in params['sa2']]
    u_tabs = []
    for lys in lys2:
        w1t = lys[0][0]
        cch = w1t.shape[1]
        ones3 = jnp.ones((1, 3), jnp.float32)
        zeros3 = jnp.zeros((1, 3), jnp.float32)
        u, _, _ = layer_pallas(c2_flat, ones3, zeros3, w1t[320:, :],
                               jnp.zeros((1, cch), jnp.float32), False,
                               blk=512)
        u_tabs.append(u)

    outs1 = []
    for g_flat, k, layers in zip(groups, ks1, params['sa1']):
        lys = _prep_layers(layers)
        g = g_flat.reshape(b * s1 * k, 6)
        ones = jnp.ones((1, 6), jnp.float32)
        zeros = jnp.zeros((1, 6), jnp.float32)
        y1, sa, sb = layer_pallas(g, ones, zeros, lys[0][0],
                                  lys[0][1][None, :], False)
        outs1.append(_mlp_tail(y1, sa, sb, lys, k))
    l1_points = jnp.concatenate(outs1, axis=-1)

    c1_flat = c1.reshape(b * s1, 3)
    r2s2 = (_r2(0.4), _r2(0.8))
    ks2 = (64, 128)
    x2 = jnp.concatenate([l1_points, c1_flat], axis=-1)
    t_tabs = []
    for lys in lys2:
        w1t, b1 = lys[0][0], lys[0][1]
        ones = jnp.ones((1, 323), jnp.float32)
        zeros = jnp.zeros((1, 323), jnp.float32)
        t, _, _ = layer_pallas(x2, ones, zeros, w1t, b1[None, :], False)
        t_tabs.append(t)
    y1s = sc_group_sa2(d2, t_tabs, u_tabs, r2s2, ks2)
    outs2 = []
    for y1, k, lys in zip(y1s, ks2, lys2):
        sa, sb = stats_pallas(y1)
        outs2.append(_mlp_tail(y1, sa, sb, lys, k))
    l2_points = jnp.concatenate(outs2, axis=-1)

    x3 = jnp.concatenate([c2_flat, l2_points], axis=-1)
    lys3 = _prep_layers(params['sa3'])
    ones = jnp.ones((1, 515), jnp.float32)
    zeros = jnp.zeros((1, 515), jnp.float32)
    y1, sa, sb = layer_pallas(x3, ones, zeros, lys3[0][0],
                              lys3[0][1][None, :], False, blk=512)
    out = _mlp_tail(y1, sa, sb, lys3, s2)
    return out


# speedup vs baseline: 1.0421x; 1.0421x over previous
import functools

import jax
import jax.numpy as jnp
import numpy as np
from jax import lax
from jax.experimental import pallas as pl
from jax.experimental.pallas import tpu as pltpu
from jax.experimental.pallas import tpu_sc as plsc

_EPS = 1e-5
_NUM_SUBCORES = 32


def _fps_body(b, n, npoint, x_ref, y_ref, z_ref, ox_ref, oy_ref, oz_ref):
    n8 = n // 8
    s8 = max(npoint // 8, 1)
    x = x_ref[...]
    y = y_ref[...]
    z = z_ref[...]
    iota_n = (lax.broadcasted_iota(jnp.int32, (b, 8, n8), 1) * n8
              + lax.broadcasted_iota(jnp.int32, (b, 8, n8), 2))
    iota_s = (lax.broadcasted_iota(jnp.int32, (b, 8, s8), 1) * s8
              + lax.broadcasted_iota(jnp.int32, (b, 8, s8), 2))

    def body(i, state):
        dist, far, ox, oy, oz = state
        sel = iota_n == far
        cx = jnp.sum(jnp.where(sel, x, 0.0), axis=(1, 2), keepdims=True)
        cy = jnp.sum(jnp.where(sel, y, 0.0), axis=(1, 2), keepdims=True)
        cz = jnp.sum(jnp.where(sel, z, 0.0), axis=(1, 2), keepdims=True)
        ox = jnp.where(iota_s == i, cx, ox)
        oy = jnp.where(iota_s == i, cy, oy)
        oz = jnp.where(iota_s == i, cz, oz)
        dx = x - cx
        dy = y - cy
        dz = z - cz
        d = (dx * dx + dy * dy) + dz * dz
        dist = jnp.minimum(dist, d)
        m = jnp.max(dist, axis=(1, 2), keepdims=True)
        far = jnp.min(jnp.where(dist == m, iota_n, n), axis=(1, 2),
                      keepdims=True)
        return dist, far, ox, oy, oz

    init = (jnp.full((b, 8, n8), 1e10, jnp.float32),
            jnp.zeros((b, 1, 1), jnp.int32),
            jnp.zeros((b, 8, s8), jnp.float32),
            jnp.zeros((b, 8, s8), jnp.float32),
            jnp.zeros((b, 8, s8), jnp.float32))
    _, _, ox, oy, oz = lax.fori_loop(0, npoint, body, init)
    ox_ref[...] = ox
    oy_ref[...] = oy
    oz_ref[...] = oz


def fps_pallas(xyz, npoint):
    b, n, _ = xyz.shape
    n8 = n // 8
    s8 = max(npoint // 8, 1)
    xr = xyz[..., 0].reshape(b, 8, n8)
    yr = xyz[..., 1].reshape(b, 8, n8)
    zr = xyz[..., 2].reshape(b, 8, n8)
    out_sh = jax.ShapeDtypeStruct((b, 8, s8), jnp.float32)
    ox, oy, oz = pl.pallas_call(
        functools.partial(_fps_body, b, n, npoint),
        out_shape=[out_sh, out_sh, out_sh],
    )(xr, yr, zr)
    return jnp.stack([ox.reshape(b, npoint), oy.reshape(b, npoint),
                      oz.reshape(b, npoint)], axis=-1)


def _sqdist_body(src_ref, dstt_ref, out_ref):
    src = src_ref[0]
    dstt = dstt_ref[0]
    sb0 = src[:, 0:1]
    sb1 = src[:, 1:2]
    sb2 = src[:, 2:3]
    ss = (sb0 * sb0 + sb1 * sb1) + sb2 * sb2
    d0 = dstt[0:1, :]
    d1 = dstt[1:2, :]
    d2 = dstt[2:3, :]
    sd = (d0 * d0 + d1 * d1) + d2 * d2
    dot = jnp.dot(src.astype(jnp.bfloat16), dstt.astype(jnp.bfloat16),
                  preferred_element_type=jnp.float32)
    out_ref[0] = (ss + sd) - 2.0 * dot


def sqdist_pallas(new_xyz, xyz, s_blk=None):
    b, s, _ = new_xyz.shape
    n = xyz.shape[1]
    if s_blk is None:
        s_blk = min(s, 256)
    xyz_t = jnp.transpose(xyz, (0, 2, 1))
    return pl.pallas_call(
        _sqdist_body,
        grid=(b, s // s_blk),
        in_specs=[pl.BlockSpec((1, s_blk, 3), lambda i, j: (i, j, 0)),
                  pl.BlockSpec((1, 3, n), lambda i, j: (i, 0, 0))],
        out_specs=pl.BlockSpec((1, s_blk, n), lambda i, j: (i, j, 0)),
        out_shape=jax.ShapeDtypeStruct((b, s, n), jnp.float32),
    )(new_xyz, xyz_t)


def _select_indices(drow_ref, idx_refs, r2s, ks, n, joff, iota16):
    nb_total = n // 16
    nbr = len(r2s)

    fill = jnp.full((16,), n - 1, jnp.int32) + joff
    for r in range(nbr):
        idx_refs[r][pl.ds(0, 16)] = fill

    def scan_blk(nb, cnts):
        dv = drow_ref[pl.ds(nb * 16, 16)]
        jv = joff + nb * 16 + iota16
        new = []
        for r in range(nbr):
            mask = dv <= r2s[r]
            cnt = cnts[r]

            @pl.when(cnt < ks[r])
            def _(idx_ref=idx_refs[r], cnt=cnt, jv=jv, mask=mask):
                plsc.store_compressed(idx_ref.at[pl.ds(cnt, 16)], jv,
                                      mask=mask)

            new.append(cnt + jnp.sum(mask.astype(jnp.int32)))
        return tuple(new)

    cnts = lax.fori_loop(0, nb_total, scan_blk,
                         tuple(jnp.int32(0) for _ in range(nbr)))

    for r in range(nbr):
        v0 = idx_refs[r][pl.ds(0, 16)]
        s0 = jnp.min(jnp.where(iota16 == 0, v0, jnp.int32(2147483647)))
        first = s0 + jnp.zeros((16,), jnp.int32)
        for p in range(ks[r] // 16):
            iv = cnts[r] + p * 16 + iota16
            plsc.store_scatter(idx_refs[r], [iv], first, mask=iv < ks[r])
    return cnts


def _sc_sa1_body(n, s, cpw, r2s, ks,
                 dd, xh, yh, zh, cxh, cyh, czh,
                 out0, out1, out2,
                 drowa, drowb, xloc, yloc, zloc, ccx, ccy, ccz,
                 idx0, idx1, idx2,
                 rb00, rb01, rb10, rb11, rb20, rb21,
                 semd0, semd1, so00, so01, so10, so11, so20, so21):
    outs = [out0, out1, out2]
    idx_refs = [idx0, idx1, idx2]
    rbs = [[rb00, rb01], [rb10, rb11], [rb20, rb21]]
    drows = [drowa, drowb]
    semd = [semd0, semd1]
    semo = [[so00, so01], [so10, so11], [so20, so21]]
    w = lax.axis_index("s") * 2 + lax.axis_index("c")
    base_c = w * cpw
    b = base_c // s
    pltpu.sync_copy(xh.at[b], xloc)
    pltpu.sync_copy(yh.at[b], yloc)
    pltpu.sync_copy(zh.at[b], zloc)
    pltpu.sync_copy(cxh.at[pl.ds(base_c, cpw)], ccx)
    pltpu.sync_copy(cyh.at[pl.ds(base_c, cpw)], ccy)
    pltpu.sync_copy(czh.at[pl.ds(base_c, cpw)], ccz)
    iota16 = lax.iota(jnp.int32, 16)
    pltpu.async_copy(dd.at[base_c], drows[0], semd[0])

    def process(c, cl2, p):
        if p == 0:
            pltpu.async_copy(dd.at[c + 1], drows[1], semd[1])
        else:
            @pl.when(cl2 + 1 < cpw // 2)
            def _():
                pltpu.async_copy(dd.at[c + 1], drows[0], semd[0])
        pltpu.make_async_copy(dd.at[c], drows[p], semd[p]).wait()
        clv = jnp.zeros((16,), jnp.int32) + (c - base_c)
        cxs = plsc.load_gather(ccx, [clv])
        cys = plsc.load_gather(ccy, [clv])
        czs = plsc.load_gather(ccz, [clv])
        _select_indices(drows[p], idx_refs, r2s, ks, n, jnp.int32(0),
                        iota16)
        for r in range(3):
            k = ks[r]
            rb = rbs[r][p]

            @pl.when(cl2 > 0)
            def _(r=r, k=k, rb=rb):
                pltpu.make_async_copy(
                    rb, outs[r].at[pl.ds((c - 2) * (k * 6), k * 6)],
                    semo[r][p]).wait()

            for q in range(k // 16):
                kv = idx_refs[r][pl.ds(q * 16, 16)]
                gx = plsc.load_gather(xloc, [kv])
                gy = plsc.load_gather(yloc, [kv])
                gz = plsc.load_gather(zloc, [kv])
                basei = (q * 16 + iota16) * 6
                plsc.store_scatter(rb, [basei], gx)
                plsc.store_scatter(rb, [basei + 1], gy)
                plsc.store_scatter(rb, [basei + 2], gz)
                plsc.store_scatter(rb, [basei + 3], gx - cxs)
                plsc.store_scatter(rb, [basei + 4], gy - cys)
                plsc.store_scatter(rb, [basei + 5], gz - czs)
            pltpu.async_copy(rb, outs[r].at[pl.ds(c * (k * 6), k * 6)],
                             semo[r][p])

    def pair(cl2, carry):
        c0 = base_c + 2 * cl2
        process(c0, cl2, 0)
        process(c0 + 1, cl2, 1)
        return carry

    lax.fori_loop(0, cpw // 2, pair, jnp.int32(0))
    for r in range(3):
        k = ks[r]
        pltpu.make_async_copy(
            rbs[r][0], outs[r].at[pl.ds((base_c + cpw - 2) * (k * 6), k * 6)],
            semo[r][0]).wait()
        pltpu.make_async_copy(
            rbs[r][1], outs[r].at[pl.ds((base_c + cpw - 1) * (k * 6), k * 6)],
            semo[r][1]).wait()


def sc_group_sa1(d, xyz, new_xyz, r2s, ks):
    bs, n = d.shape
    b3 = xyz.shape[0]
    s = bs // b3
    cpw = bs // _NUM_SUBCORES
    mesh = plsc.VectorSubcoreMesh(core_axis_name="c", subcore_axis_name="s")
    out_type = [jax.ShapeDtypeStruct((bs * k * 6,), jnp.float32) for k in ks]
    scratch = [
        pltpu.VMEM((n,), jnp.float32),
        pltpu.VMEM((n,), jnp.float32),
        pltpu.VMEM((n,), jnp.float32),
        pltpu.VMEM((n,), jnp.float32),
        pltpu.VMEM((n,), jnp.float32),
        pltpu.VMEM((cpw,), jnp.float32),
        pltpu.VMEM((cpw,), jnp.float32),
        pltpu.VMEM((cpw,), jnp.float32),
        pltpu.VMEM((ks[0] + 16,), jnp.int32),
        pltpu.VMEM((ks[1] + 16,), jnp.int32),
        pltpu.VMEM((ks[2] + 16,), jnp.int32),
        pltpu.VMEM((ks[0] * 6,), jnp.float32),
        pltpu.VMEM((ks[0] * 6,), jnp.float32),
        pltpu.VMEM((ks[1] * 6,), jnp.float32),
        pltpu.VMEM((ks[1] * 6,), jnp.float32),
        pltpu.VMEM((ks[2] * 6,), jnp.float32),
        pltpu.VMEM((ks[2] * 6,), jnp.float32),
        pltpu.SemaphoreType.DMA,
        pltpu.SemaphoreType.DMA,
        pltpu.SemaphoreType.DMA,
        pltpu.SemaphoreType.DMA,
        pltpu.SemaphoreType.DMA,
        pltpu.SemaphoreType.DMA,
        pltpu.SemaphoreType.DMA,
        pltpu.SemaphoreType.DMA,
    ]
    fn = pl.kernel(
        functools.partial(_sc_sa1_body, n, s, cpw, r2s, ks),
        out_type=out_type, mesh=mesh, scratch_types=scratch,
        compiler_params=pltpu.CompilerParams(needs_layout_passes=False))
    return fn(d,
              xyz[..., 0], xyz[..., 1], xyz[..., 2],
              new_xyz[:, 0], new_xyz[:, 1], new_xyz[:, 2])


def _sc_sa2_body(n, s, cpw, c1, r2s, ks,
                 dd, t0, t1, u0, u1,
                 y0, y1,
                 drowa, drowb, idx0, idx1, idxk0, idxk1, uloc0, uloc1,
                 rows00, rows01, rows10, rows11,
                 semd0, semd1, semg, so00, so01, so10, so11):
    ts = [t0, t1]
    ys = [y0, y1]
    idx_refs = [idx0, idx1]
    idxk_refs = [idxk0, idxk1]
    ulocs = [uloc0, uloc1]
    rows = [[rows00, rows01], [rows10, rows11]]
    drows = [drowa, drowb]
    semd = [semd0, semd1]
    semo = [[so00, so01], [so10, so11]]
    w = lax.axis_index("s") * 2 + lax.axis_index("c")
    base_c = w * cpw
    iota16 = lax.iota(jnp.int32, 16)
    nch = c1 // 16
    pltpu.sync_copy(u0.at[pl.ds(base_c, cpw)], uloc0)
    pltpu.sync_copy(u1.at[pl.ds(base_c, cpw)], uloc1)
    pltpu.async_copy(dd.at[base_c], drows[0], semd[0])

    def process(c, cl2, p):
        if p == 0:
            pltpu.async_copy(dd.at[c + 1], drows[1], semd[1])
        else:
            @pl.when(cl2 + 1 < cpw // 2)
            def _():
                pltpu.async_copy(dd.at[c + 1], drows[0], semd[0])
        pltpu.make_async_copy(dd.at[c], drows[p], semd[p]).wait()
        joff = (c // s) * n
        cl = c - base_c
        _select_indices(drows[p], idx_refs, r2s, ks, n, joff, iota16)
        for r in range(2):
            k = ks[r]
            rw = rows[r][p]
            for q in range(k // 16):
                idxk_refs[r][pl.ds(q * 16, 16)] = \
                    idx_refs[r][pl.ds(q * 16, 16)]

            @pl.when(cl2 > 0)
            def _(r=r, k=k, rw=rw):
                pltpu.make_async_copy(
                    rw, ys[r].at[pl.ds((c - 2) * k, k)], semo[r][p]).wait()

            pltpu.async_copy(ts[r].at[idxk_refs[r]], rw, semg).wait()
            ur = ulocs[r].at[cl]

            def sub_row(kk, carry2, rw=rw, ur=ur):
                rr = rw.at[kk]
                for ch in range(nch):
                    sl = pl.ds(ch * 16, 16)
                    rr[sl] = rr[sl] - ur[sl]
                return carry2

            lax.fori_loop(0, k, sub_row, jnp.int32(0))
            pltpu.async_copy(rw, ys[r].at[pl.ds(c * k, k)], semo[r][p])

    def pair(cl2, carry):
        c0 = base_c + 2 * cl2
        process(c0, cl2, 0)
        process(c0 + 1, cl2, 1)
        return carry

    lax.fori_loop(0, cpw // 2, pair, jnp.int32(0))
    for r in range(2):
        k = ks[r]
        pltpu.make_async_copy(
            rows[r][0], ys[r].at[pl.ds((base_c + cpw - 2) * k, k)],
            semo[r][0]).wait()
        pltpu.make_async_copy(
            rows[r][1], ys[r].at[pl.ds((base_c + cpw - 1) * k, k)],
            semo[r][1]).wait()


def sc_group_sa2(d, t_tabs, u_tabs, r2s, ks):
    bs, n = d.shape
    c1 = t_tabs[0].shape[1]
    bt = t_tabs[0].shape[0] // n
    s = bs // bt
    cpw = bs // _NUM_SUBCORES
    mesh = plsc.VectorSubcoreMesh(core_axis_name="c", subcore_axis_name="s")
    out_type = [jax.ShapeDtypeStruct((bs * k, c1), jnp.float32) for k in ks]
    scratch = [
        pltpu.VMEM((n,), jnp.float32),
        pltpu.VMEM((n,), jnp.float32),
        pltpu.VMEM((ks[0] + 16,), jnp.int32),
        pltpu.VMEM((ks[1] + 16,), jnp.int32),
        pltpu.VMEM((ks[0],), jnp.int32),
        pltpu.VMEM((ks[1],), jnp.int32),
        pltpu.VMEM((cpw, c1), jnp.float32),
        pltpu.VMEM((cpw, c1), jnp.float32),
        pltpu.VMEM((ks[0], c1), jnp.float32),
        pltpu.VMEM((ks[0], c1), jnp.float32),
        pltpu.VMEM((ks[1], c1), jnp.float32),
        pltpu.VMEM((ks[1], c1), jnp.float32),
        pltpu.SemaphoreType.DMA,
        pltpu.SemaphoreType.DMA,
        pltpu.SemaphoreType.DMA,
        pltpu.SemaphoreType.DMA,
        pltpu.SemaphoreType.DMA,
        pltpu.SemaphoreType.DMA,
        pltpu.SemaphoreType.DMA,
    ]
    fn = pl.kernel(
        functools.partial(_sc_sa2_body, n, s, cpw, c1, r2s, ks),
        out_type=out_type, mesh=mesh, scratch_types=scratch,
        compiler_params=pltpu.CompilerParams(needs_layout_passes=False))
    return fn(d, t_tabs[0], t_tabs[1], u_tabs[0], u_tabs[1])


def _layer_body(apply_relu, x_ref, a_ref, d_ref, wt_ref, b_ref,
                y_ref, s1_ref, s2_ref):
    x = x_ref[...]
    x = x * a_ref[...] + d_ref[...]
    if apply_relu:
        x = jnp.maximum(x, 0.0)
    y = jnp.dot(x.astype(jnp.bfloat16), wt_ref[...].astype(jnp.bfloat16),
                preferred_element_type=jnp.float32) + b_ref[...]
    y_ref[...] = y
    s1 = jnp.sum(y, axis=0, keepdims=True)
    s2 = jnp.sum(y * y, axis=0, keepdims=True)

    @pl.when(pl.program_id(0) == 0)
    def _():
        s1_ref[...] = s1
        s2_ref[...] = s2

    @pl.when(pl.program_id(0) != 0)
    def _():
        s1_ref[...] = s1_ref[...] + s1
        s2_ref[...] = s2_ref[...] + s2


def layer_pallas(x, a, d, wt, bias, apply_relu, blk=2048):
    r, cin = x.shape
    cout = wt.shape[1]
    blk = min(blk, r)
    assert r % blk == 0
    grid = r // blk
    return pl.pallas_call(
        functools.partial(_layer_body, apply_relu),
        grid=(grid,),
        in_specs=[pl.BlockSpec((blk, cin), lambda i: (i, 0)),
                  pl.BlockSpec((1, cin), lambda i: (0, 0)),
                  pl.BlockSpec((1, cin), lambda i: (0, 0)),
                  pl.BlockSpec((cin, cout), lambda i: (0, 0)),
                  pl.BlockSpec((1, cout), lambda i: (0, 0))],
        out_specs=[pl.BlockSpec((blk, cout), lambda i: (i, 0)),
                   pl.BlockSpec((1, cout), lambda i: (0, 0)),
                   pl.BlockSpec((1, cout), lambda i: (0, 0))],
        out_shape=[jax.ShapeDtypeStruct((r, cout), jnp.float32),
                   jax.ShapeDtypeStruct((1, cout), jnp.float32),
                   jax.ShapeDtypeStruct((1, cout), jnp.float32)],
    )(x, a, d, wt, bias)


def _stats_body(y_ref, s1_ref, s2_ref):
    y = y_ref[...]
    s1 = jnp.sum(y, axis=0, keepdims=True)
    s2 = jnp.sum(y * y, axis=0, keepdims=True)

    @pl.when(pl.program_id(0) == 0)
    def _():
        s1_ref[...] = s1
        s2_ref[...] = s2

    @pl.when(pl.program_id(0) != 0)
    def _():
        s1_ref[...] = s1_ref[...] + s1
        s2_ref[...] = s2_ref[...] + s2


def stats_pallas(y, blk=2048):
    r, c = y.shape
    blk = min(blk, r)
    grid = r // blk
    return pl.pallas_call(
        _stats_body,
        grid=(grid,),
        in_specs=[pl.BlockSpec((blk, c), lambda i: (i, 0))],
        out_specs=[pl.BlockSpec((1, c), lambda i: (0, 0)),
                   pl.BlockSpec((1, c), lambda i: (0, 0))],
        out_shape=[jax.ShapeDtypeStruct((1, c), jnp.float32),
                   jax.ShapeDtypeStruct((1, c), jnp.float32)],
    )(y)


def _maxpool_body(m, k, y_ref, a_ref, d_ref, out_ref):
    x = y_ref[...] * a_ref[...] + d_ref[...]
    x = jnp.maximum(x, 0.0)
    x = x.reshape(m, k, x.shape[-1])
    out_ref[...] = jnp.max(x, axis=1)


def maxpool_pallas(y, a, d, k, blk_rows=2048):
    r, c = y.shape
    blk_rows = min(blk_rows, r)
    m = blk_rows // k
    grid = r // blk_rows
    return pl.pallas_call(
        functools.partial(_maxpool_body, m, k),
        grid=(grid,),
        in_specs=[pl.BlockSpec((blk_rows, c), lambda i: (i, 0)),
                  pl.BlockSpec((1, c), lambda i: (0, 0)),
                  pl.BlockSpec((1, c), lambda i: (0, 0))],
        out_specs=pl.BlockSpec((m, c), lambda i: (i, 0)),
        out_shape=jax.ShapeDtypeStruct((r // k, c), jnp.float32),
    )(y, a, d)


def _fold_bn(s1, s2, r, gamma, beta):
    mean = s1[0] / r
    var = jnp.maximum(s2[0] / r - mean * mean, 0.0)
    a = gamma / jnp.sqrt(var + _EPS)
    d = beta - mean * a
    return a[None, :], d[None, :]


def _mlp_tail(y1, s1a, s1b, layers, k):
    r = y1.shape[0]
    y, sa, sb = y1, s1a, s1b
    for i in range(1, len(layers)):
        a, d = _fold_bn(sa, sb, r, layers[i - 1][2], layers[i - 1][3])
        y, sa, sb = layer_pallas(y, a, d, layers[i][0], layers[i][1][None, :],
                                 True)
    a, d = _fold_bn(sa, sb, r, layers[-1][2], layers[-1][3])
    return maxpool_pallas(y, a, d, k)


def _prep_layers(layers):
    return [(w.T, b, g, bt) for (w, b, g, bt) in layers]


def _r2(radius):
    return np.float32(np.float64(radius) ** 2)


def kernel(xyz, params):
    b, n, _ = xyz.shape
    s1 = 512
    s2 = 128

    c1 = fps_pallas(xyz, s1)
    d1 = sqdist_pallas(c1, xyz).reshape(b * s1, n)
    r2s1 = (_r2(0.1), _r2(0.2), _r2(0.4))
    ks1 = (32, 64, 128)
    groups = sc_group_sa1(d1, xyz, c1.reshape(b * s1, 3), r2s1, ks1)

    c2 = fps_pallas(c1, s2)
    d2 = sqdist_pallas(c2, c1).reshape(b * s2, s1)
    c2_flat = c2.reshape(b * s2, 3)
    lys2 = [_prep_layers(layers) for layers in params['sa2']]
    u_tabs = []
    for lys in lys2:
        w1t = lys[0][0]
        cch = w1t.shape[1]
        ones3 = jnp.ones((1, 3), jnp.float32)
        zeros3 = jnp.zeros((1, 3), jnp.float32)
        u, _, _ = layer_pallas(c2_flat, ones3, zeros3, w1t[320:, :],
                               jnp.zeros((1, cch), jnp.float32), False,
                               blk=512)
        u_tabs.append(u)

    outs1 = []
    for g_flat, k, layers in zip(groups, ks1, params['sa1']):
        lys = _prep_layers(layers)
        g = g_flat.reshape(b * s1 * k, 6)
        ones = jnp.ones((1, 6), jnp.float32)
        zeros = jnp.zeros((1, 6), jnp.float32)
        y1, sa, sb = layer_pallas(g, ones, zeros, lys[0][0],
                                  lys[0][1][None, :], False)
        outs1.append(_mlp_tail(y1, sa, sb, lys, k))
    l1_points = jnp.concatenate(outs1, axis=-1)

    c1_flat = c1.reshape(b * s1, 3)
    r2s2 = (_r2(0.4), _r2(0.8))
    ks2 = (64, 128)
    x2 = jnp.concatenate([l1_points, c1_flat], axis=-1)
    t_tabs = []
    for lys in lys2:
        w1t, b1 = lys[0][0], lys[0][1]
        ones = jnp.ones((1, 323), jnp.float32)
        zeros = jnp.zeros((1, 323), jnp.float32)
        t, _, _ = layer_pallas(x2, ones, zeros, w1t, b1[None, :], False)
        t_tabs.append(t)
    y1s = sc_group_sa2(d2, t_tabs, u_tabs, r2s2, ks2)
    outs2 = []
    for y1, k, lys in zip(y1s, ks2, lys2):
        sa, sb = stats_pallas(y1)
        outs2.append(_mlp_tail(y1, sa, sb, lys, k))
    l2_points = jnp.concatenate(outs2, axis=-1)

    x3 = jnp.concatenate([c2_flat, l2_points], axis=-1)
    lys3 = _prep_layers(params['sa3'])
    ones = jnp.ones((1, 515), jnp.float32)
    zeros = jnp.zeros((1, 515), jnp.float32)
    y1, sa, sb = layer_pallas(x3, ones, zeros, lys3[0][0],
                              lys3[0][1][None, :], False, blk=512)
    out = _mlp_tail(y1, sa, sb, lys3, s2)
    return out
